# baseline (speedup 1.0000x reference)
import functools

import jax
import jax.numpy as jnp
from jax import lax
from jax.experimental import pallas as pl

ICP_ITERS = 5
HUBER_DELTA = 1.0
TRIM_DIST = 5.0
BFAR_TEMP = 10.0
CHUNK = 4096


def _icp_kernel(scanT_ref, inten_ref, map_ref, mapT_ref, T0_ref, params_ref,
                out_ref, *, n_pts, n_map):
    N = n_pts
    M = n_map
    C = CHUNK
    n_chunks = M // C

    scanTb = scanT_ref[0]
    inten = inten_ref[0]
    T = T0_ref[0]
    prm = params_ref[...]

    a = jnp.maximum(prm[0, 0], 0.0)
    b = jnp.maximum(prm[0, 1], 0.0)
    thresh = a * jnp.mean(inten) + b
    w_bfar = jax.nn.sigmoid((inten - thresh) * BFAR_TEMP)

    for _ in range(ICP_ITERS):
        Rb = T[:3, :3].astype(jnp.bfloat16)
        sT = lax.dot_general(Rb, scanTb, (((1,), (0,)), ((), ())),
                             preferred_element_type=jnp.float32)
        sx = sT[0:1, :] + T[0, 3]
        sy = sT[1:2, :] + T[1, 3]
        sz = sT[2:3, :] + T[2, 3]

        def chunk_body(ci, carry, sx=sx, sy=sy, sz=sz):
            run_min, run_arg = carry
            m = map_ref[0, pl.ds(ci * C, C), :]
            dx = m[:, 0:1] - sx
            dy = m[:, 1:2] - sy
            dz = m[:, 2:3] - sz
            d2 = dx * dx + dy * dy + dz * dz
            tmin = jnp.min(d2, axis=0, keepdims=True)
            iota = lax.broadcasted_iota(jnp.int32, (C, N), 0) + ci * C
            targ = jnp.min(jnp.where(d2 == tmin, iota, M),
                           axis=0, keepdims=True)
            better = tmin < run_min
            return (jnp.where(better, tmin, run_min),
                    jnp.where(better, targ, run_arg))

        run_min0 = jnp.full((1, N), jnp.inf, dtype=jnp.float32)
        run_arg0 = jnp.zeros((1, N), dtype=jnp.int32)
        _, idx = lax.fori_loop(0, n_chunks, chunk_body, (run_min0, run_arg0))

        def gather_body(ci, nn, idx=idx):
            iota = lax.broadcasted_iota(jnp.int32, (C, N), 0) + ci * C
            onehot = (iota == idx).astype(jnp.float32)
            mT = mapT_ref[0, :, pl.ds(ci * C, C)]
            return nn + lax.dot_general(
                mT, onehot, (((1,), (0,)), ((), ())),
                preferred_element_type=jnp.float32)

        nn = lax.fori_loop(0, n_chunks, gather_body,
                           jnp.zeros((3, N), dtype=jnp.float32))

        nx = nn[0:1, :]
        ny = nn[1:2, :]
        nz = nn[2:3, :]
        rx = nx - sx
        ry = ny - sy
        rz = nz - sz
        d2r = rx * rx + ry * ry + rz * rz
        dist = jnp.sqrt(d2r + 1e-12)
        w_h = jnp.where(dist < HUBER_DELTA, 1.0, HUBER_DELTA / dist)
        w = w_bfar * w_h * (dist < TRIM_DIST).astype(jnp.float32)
        wsum = jnp.sum(w) + 1e-8

        mu_sx = jnp.sum(w * sx) / wsum
        mu_sy = jnp.sum(w * sy) / wsum
        mu_mx = jnp.sum(w * nx) / wsum
        mu_my = jnp.sum(w * ny) / wsum
        sc0 = sx - mu_sx
        sc1 = sy - mu_sy
        mc0 = nx - mu_mx
        mc1 = ny - mu_my
        cross = jnp.sum(w * (sc0 * mc1 - sc1 * mc0))
        dot = jnp.sum(w * (sc0 * mc0 + sc1 * mc1))
        h = jnp.sqrt(cross * cross + dot * dot)
        safe = h > 0.0
        c = jnp.where(safe, dot / jnp.where(safe, h, 1.0), 1.0)
        sn = jnp.where(safe, cross / jnp.where(safe, h, 1.0), 0.0)
        t2x = mu_mx - (c * mu_sx - sn * mu_sy)
        t2y = mu_my - (sn * mu_sx + c * mu_sy)

        row0 = c * T[0:1, :] - sn * T[1:2, :] + t2x * T[3:4, :]
        row1 = sn * T[0:1, :] + c * T[1:2, :] + t2y * T[3:4, :]
        T = jnp.concatenate([row0, row1, T[2:3, :], T[3:4, :]], axis=0)

    out_ref[0] = T


def kernel(scan_pc, scan_intensity, map_pc, T_init, params):
    B, N, _ = scan_pc.shape
    M = map_pc.shape[1]
    scanT = jnp.transpose(scan_pc, (0, 2, 1)).astype(jnp.bfloat16)
    mapT = jnp.transpose(map_pc, (0, 2, 1))
    inten3 = scan_intensity[:, None, :]
    prm2 = params.reshape(1, 2)

    f = functools.partial(_icp_kernel, n_pts=N, n_map=M)
    return pl.pallas_call(
        f,
        grid=(B,),
        in_specs=[
            pl.BlockSpec((1, 3, N), lambda i: (i, 0, 0)),
            pl.BlockSpec((1, 1, N), lambda i: (i, 0, 0)),
            pl.BlockSpec((1, M, 3), lambda i: (i, 0, 0)),
            pl.BlockSpec((1, 3, M), lambda i: (i, 0, 0)),
            pl.BlockSpec((1, 4, 4), lambda i: (i, 0, 0)),
            pl.BlockSpec((1, 2), lambda i: (0, 0)),
        ],
        out_specs=pl.BlockSpec((1, 4, 4), lambda i: (i, 0, 0)),
        out_shape=jax.ShapeDtypeStruct((B, 4, 4), jnp.float32),
    )(scanT, inten3, map_pc, mapT, T_init, prm2)


# speedup vs baseline: 1.1013x; 1.1013x over previous
import functools

import jax
import jax.numpy as jnp
from jax import lax
from jax.experimental import pallas as pl
from jax.experimental.pallas import tpu as pltpu
from jax.experimental.pallas import tpu_sc as plsc

ICP_ITERS = 5
HUBER_DELTA = 1.0
TRIM_DIST = 5.0
BFAR_TEMP = 10.0
CHUNK = 2048

M_SC = 6144
NC, NS, L = 2, 16, 16
NW = NC * NS
SC_QT = 4
SC_UNROLL = 2

_sc_mesh = plsc.VectorSubcoreMesh(core_axis_name="c", subcore_axis_name="s")


def _make_sc_nn(n_queries, m_sc, n_per_batch):
    qpw = n_queries // NW

    @functools.partial(
        pl.kernel, mesh=_sc_mesh,
        out_type=[jax.ShapeDtypeStruct((n_queries, L), jnp.float32),
                  jax.ShapeDtypeStruct((n_queries, L), jnp.int32)],
        scratch_types=[
            pltpu.VMEM((m_sc,), jnp.float32),
            pltpu.VMEM((m_sc,), jnp.float32),
            pltpu.VMEM((m_sc,), jnp.float32),
            pltpu.VMEM((qpw, L), jnp.float32),
            pltpu.VMEM((qpw, L), jnp.float32),
            pltpu.VMEM((qpw, L), jnp.float32),
            pltpu.VMEM((qpw, L), jnp.float32),
            pltpu.VMEM((qpw, L), jnp.int32),
        ],
    )
    def sc_nn(qxb_h, qyb_h, qzb_h, mx_h, my_h, mz_h, omin_h, oidx_h,
              mxv, myv, mzv, qxv, qyv, qzv, rminv, ridxv):
        wid = lax.axis_index("s") * NC + lax.axis_index("c")
        qbase = wid * qpw
        batch = qbase // n_per_batch
        mbase = batch * m_sc
        pltpu.sync_copy(mx_h.at[pl.ds(mbase, m_sc)], mxv)
        pltpu.sync_copy(my_h.at[pl.ds(mbase, m_sc)], myv)
        pltpu.sync_copy(mz_h.at[pl.ds(mbase, m_sc)], mzv)
        pltpu.sync_copy(qxb_h.at[pl.ds(qbase, qpw)], qxv)
        pltpu.sync_copy(qyb_h.at[pl.ds(qbase, qpw)], qyv)
        pltpu.sync_copy(qzb_h.at[pl.ds(qbase, qpw)], qzv)

        lane = lax.broadcasted_iota(jnp.int32, (L,), 0)

        for qt in range(qpw // SC_QT):
            qx = [qxv[qt * SC_QT + j, :] for j in range(SC_QT)]
            qy = [qyv[qt * SC_QT + j, :] for j in range(SC_QT)]
            qz = [qzv[qt * SC_QT + j, :] for j in range(SC_QT)]

            def body(k, carry, qx=qx, qy=qy, qz=qz):
                bmins, bidxs = carry
                bmins = list(bmins)
                bidxs = list(bidxs)
                for u in range(SC_UNROLL):
                    base = (k * SC_UNROLL + u) * L
                    mx = mxv[pl.ds(base, L)]
                    my = myv[pl.ds(base, L)]
                    mz = mzv[pl.ds(base, L)]
                    idxvec = lane + base
                    for j in range(SC_QT):
                        dx = mx - qx[j]
                        dy = my - qy[j]
                        dz = mz - qz[j]
                        d2 = dx * dx + dy * dy + dz * dz
                        better = d2 < bmins[j]
                        bmins[j] = jnp.where(better, d2, bmins[j])
                        bidxs[j] = jnp.where(better, idxvec, bidxs[j])
                return tuple(bmins), tuple(bidxs)

            inf16 = jnp.full((L,), jnp.inf, dtype=jnp.float32)
            z16 = jnp.zeros((L,), dtype=jnp.int32)
            bmins, bidxs = lax.fori_loop(
                0, m_sc // (L * SC_UNROLL), body,
                (tuple(inf16 for _ in range(SC_QT)),
                 tuple(z16 for _ in range(SC_QT))))

            for j in range(SC_QT):
                rminv[qt * SC_QT + j, :] = bmins[j]
                ridxv[qt * SC_QT + j, :] = bidxs[j]

        pltpu.sync_copy(rminv, omin_h.at[pl.ds(qbase, qpw)])
        pltpu.sync_copy(ridxv, oidx_h.at[pl.ds(qbase, qpw)])

    return sc_nn


def _tc_dist_kernel(sT_ref, map_ref, min_ref, arg_ref, *, n_pts, m_tc):
    N = n_pts
    C = CHUNK
    sT = sT_ref[0]
    sx = sT[0:1, :]
    sy = sT[1:2, :]
    sz = sT[2:3, :]

    def chunk_body(ci, carry):
        run_min, run_arg = carry
        m = map_ref[0, pl.ds(ci * C, C), :]
        dx = m[:, 0:1] - sx
        dy = m[:, 1:2] - sy
        dz = m[:, 2:3] - sz
        d2 = dx * dx + dy * dy + dz * dz
        tmin = jnp.min(d2, axis=0, keepdims=True)
        iota = lax.broadcasted_iota(jnp.int32, (C, N), 0) + ci * C
        targ = jnp.min(jnp.where(d2 == tmin, iota, m_tc),
                       axis=0, keepdims=True)
        better = tmin < run_min
        return (jnp.where(better, tmin, run_min),
                jnp.where(better, targ, run_arg))

    run_min0 = jnp.full((1, N), jnp.inf, dtype=jnp.float32)
    run_arg0 = jnp.zeros((1, N), dtype=jnp.int32)
    rmin, rarg = lax.fori_loop(0, m_tc // C, chunk_body, (run_min0, run_arg0))
    min_ref[0] = rmin
    arg_ref[0] = rarg


def _tc_stats_kernel(sT_ref, inten_ref, mapT_ref, min1_ref, arg1_ref,
                     scmin_ref, scidx_ref, T_ref, params_ref, out_ref,
                     *, n_pts, n_map, m_tc):
    N = n_pts
    M = n_map
    C = CHUNK

    sT = sT_ref[0]
    inten = inten_ref[0]
    T = T_ref[0]
    prm = params_ref[...]

    a = jnp.maximum(prm[0, 0], 0.0)
    b = jnp.maximum(prm[0, 1], 0.0)
    thresh = a * jnp.mean(inten) + b
    w_bfar = jax.nn.sigmoid((inten - thresh) * BFAR_TEMP)

    sx = sT[0:1, :]
    sy = sT[1:2, :]
    sz = sT[2:3, :]

    pm = scmin_ref[0]
    pi = scidx_ref[0]
    gmin2 = jnp.min(pm, axis=1, keepdims=True)
    cand = jnp.where(pm == gmin2, pi, M)
    gidx2 = jnp.min(cand, axis=1, keepdims=True)
    m2 = jnp.transpose(gmin2)
    a2 = jnp.transpose(gidx2) + m_tc

    m1 = min1_ref[0]
    a1 = arg1_ref[0]
    take2 = (m2 < m1) | ((m2 == m1) & (a2 < a1))
    idx = jnp.where(take2, a2, a1)

    def gather_body(ci, nn, idx=idx):
        iota = lax.broadcasted_iota(jnp.int32, (C, N), 0) + ci * C
        onehot = (iota == idx).astype(jnp.float32)
        mT = mapT_ref[0, :, pl.ds(ci * C, C)]
        return nn + lax.dot_general(
            mT, onehot, (((1,), (0,)), ((), ())),
            preferred_element_type=jnp.float32)

    nn = lax.fori_loop(0, M // C, gather_body,
                       jnp.zeros((3, N), dtype=jnp.float32))

    nx = nn[0:1, :]
    ny = nn[1:2, :]
    nz = nn[2:3, :]
    rx = nx - sx
    ry = ny - sy
    rz = nz - sz
    d2r = rx * rx + ry * ry + rz * rz
    dist = jnp.sqrt(d2r + 1e-12)
    w_h = jnp.where(dist < HUBER_DELTA, 1.0, HUBER_DELTA / dist)
    w = w_bfar * w_h * (dist < TRIM_DIST).astype(jnp.float32)
    wsum = jnp.sum(w) + 1e-8

    mu_sx = jnp.sum(w * sx) / wsum
    mu_sy = jnp.sum(w * sy) / wsum
    mu_mx = jnp.sum(w * nx) / wsum
    mu_my = jnp.sum(w * ny) / wsum
    sc0 = sx - mu_sx
    sc1 = sy - mu_sy
    mc0 = nx - mu_mx
    mc1 = ny - mu_my
    cross = jnp.sum(w * (sc0 * mc1 - sc1 * mc0))
    dot = jnp.sum(w * (sc0 * mc0 + sc1 * mc1))
    h = jnp.sqrt(cross * cross + dot * dot)
    safe = h > 0.0
    c = jnp.where(safe, dot / jnp.where(safe, h, 1.0), 1.0)
    sn = jnp.where(safe, cross / jnp.where(safe, h, 1.0), 0.0)
    t2x = mu_mx - (c * mu_sx - sn * mu_sy)
    t2y = mu_my - (sn * mu_sx + c * mu_sy)

    row0 = c * T[0:1, :] - sn * T[1:2, :] + t2x * T[3:4, :]
    row1 = sn * T[0:1, :] + c * T[1:2, :] + t2y * T[3:4, :]
    out_ref[0] = jnp.concatenate([row0, row1, T[2:3, :], T[3:4, :]], axis=0)


def kernel(scan_pc, scan_intensity, map_pc, T_init, params):
    B, N, _ = scan_pc.shape
    M = map_pc.shape[1]
    m_tc = M - M_SC
    NQ = B * N

    mapT = jnp.transpose(map_pc, (0, 2, 1))
    map_tc = map_pc[:, :m_tc, :]
    sc_mx = mapT[:, 0, m_tc:].reshape(-1)
    sc_my = mapT[:, 1, m_tc:].reshape(-1)
    sc_mz = mapT[:, 2, m_tc:].reshape(-1)
    inten3 = scan_intensity[:, None, :]
    prm2 = params.reshape(1, 2)

    sc_nn = _make_sc_nn(NQ, M_SC, N)

    dist_call = pl.pallas_call(
        functools.partial(_tc_dist_kernel, n_pts=N, m_tc=m_tc),
        grid=(B,),
        in_specs=[
            pl.BlockSpec((1, 3, N), lambda i: (i, 0, 0)),
            pl.BlockSpec((1, m_tc, 3), lambda i: (i, 0, 0)),
        ],
        out_specs=[
            pl.BlockSpec((1, 1, N), lambda i: (i, 0, 0)),
            pl.BlockSpec((1, 1, N), lambda i: (i, 0, 0)),
        ],
        out_shape=[
            jax.ShapeDtypeStruct((B, 1, N), jnp.float32),
            jax.ShapeDtypeStruct((B, 1, N), jnp.int32),
        ],
    )

    stats_call = pl.pallas_call(
        functools.partial(_tc_stats_kernel, n_pts=N, n_map=M, m_tc=m_tc),
        grid=(B,),
        in_specs=[
            pl.BlockSpec((1, 3, N), lambda i: (i, 0, 0)),
            pl.BlockSpec((1, 1, N), lambda i: (i, 0, 0)),
            pl.BlockSpec((1, 3, M), lambda i: (i, 0, 0)),
            pl.BlockSpec((1, 1, N), lambda i: (i, 0, 0)),
            pl.BlockSpec((1, 1, N), lambda i: (i, 0, 0)),
            pl.BlockSpec((1, N, L), lambda i: (i, 0, 0)),
            pl.BlockSpec((1, N, L), lambda i: (i, 0, 0)),
            pl.BlockSpec((1, 4, 4), lambda i: (i, 0, 0)),
            pl.BlockSpec((1, 2), lambda i: (0, 0)),
        ],
        out_specs=pl.BlockSpec((1, 4, 4), lambda i: (i, 0, 0)),
        out_shape=jax.ShapeDtypeStruct((B, 4, 4), jnp.float32),
    )

    T = T_init
    for _ in range(ICP_ITERS):
        R = T[:, :3, :3]
        t = T[:, :3, 3]
        s = jnp.einsum('bnk,bjk->bnj', scan_pc, R) + t[:, None, :]
        sT = jnp.transpose(s, (0, 2, 1))
        sq = s.reshape(NQ, 3, 1)
        qb = jnp.broadcast_to(sq, (NQ, 3, L))
        qxb = qb[:, 0, :]
        qyb = qb[:, 1, :]
        qzb = qb[:, 2, :]

        pmin, pidx = sc_nn(qxb, qyb, qzb, sc_mx, sc_my, sc_mz)
        min1, arg1 = dist_call(sT, map_tc)
        T = stats_call(sT, inten3, mapT, min1, arg1,
                       pmin.reshape(B, N, L), pidx.reshape(B, N, L),
                       T, prm2)
    return T


# speedup vs baseline: 1.1665x; 1.0592x over previous
import functools

import jax
import jax.numpy as jnp
from jax import lax
from jax.experimental import pallas as pl
from jax.experimental.pallas import tpu as pltpu
from jax.experimental.pallas import tpu_sc as plsc

ICP_ITERS = 5
HUBER_DELTA = 1.0
TRIM_DIST = 5.0
BFAR_TEMP = 10.0
CHUNK = 2048

M_SC = 5632
NC, NS, L = 2, 16, 16
NW = NC * NS
SC_QT = 4
SC_UNROLL = 2

_sc_mesh = plsc.VectorSubcoreMesh(core_axis_name="c", subcore_axis_name="s")


def _make_sc_nn(n_queries, m_sc, n_per_batch):
    qpw = n_queries // NW

    @functools.partial(
        pl.kernel, mesh=_sc_mesh,
        out_type=[jax.ShapeDtypeStruct((n_queries, L), jnp.float32),
                  jax.ShapeDtypeStruct((n_queries, L), jnp.int32)],
        scratch_types=[
            pltpu.VMEM((m_sc,), jnp.float32),
            pltpu.VMEM((m_sc,), jnp.float32),
            pltpu.VMEM((m_sc,), jnp.float32),
            pltpu.VMEM((qpw, L), jnp.float32),
            pltpu.VMEM((qpw, L), jnp.float32),
            pltpu.VMEM((qpw, L), jnp.float32),
            pltpu.VMEM((qpw, L), jnp.float32),
            pltpu.VMEM((qpw, L), jnp.int32),
        ],
    )
    def sc_nn(qxb_h, qyb_h, qzb_h, mx_h, my_h, mz_h, omin_h, oidx_h,
              mxv, myv, mzv, qxv, qyv, qzv, rminv, ridxv):
        wid = lax.axis_index("s") * NC + lax.axis_index("c")
        qbase = wid * qpw
        batch = qbase // n_per_batch
        mbase = batch * m_sc
        pltpu.sync_copy(mx_h.at[pl.ds(mbase, m_sc)], mxv)
        pltpu.sync_copy(my_h.at[pl.ds(mbase, m_sc)], myv)
        pltpu.sync_copy(mz_h.at[pl.ds(mbase, m_sc)], mzv)
        pltpu.sync_copy(qxb_h.at[pl.ds(qbase, qpw)], qxv)
        pltpu.sync_copy(qyb_h.at[pl.ds(qbase, qpw)], qyv)
        pltpu.sync_copy(qzb_h.at[pl.ds(qbase, qpw)], qzv)

        lane = lax.broadcasted_iota(jnp.int32, (L,), 0)

        for qt in range(qpw // SC_QT):
            qx = [qxv[qt * SC_QT + j, :] for j in range(SC_QT)]
            qy = [qyv[qt * SC_QT + j, :] for j in range(SC_QT)]
            qz = [qzv[qt * SC_QT + j, :] for j in range(SC_QT)]

            def body(k, carry, qx=qx, qy=qy, qz=qz):
                bmins, bidxs = carry
                bmins = list(bmins)
                bidxs = list(bidxs)
                for u in range(SC_UNROLL):
                    base = (k * SC_UNROLL + u) * L
                    mx = mxv[pl.ds(base, L)]
                    my = myv[pl.ds(base, L)]
                    mz = mzv[pl.ds(base, L)]
                    idxvec = lane + base
                    for j in range(SC_QT):
                        dx = mx - qx[j]
                        dy = my - qy[j]
                        dz = mz - qz[j]
                        d2 = dx * dx + dy * dy + dz * dz
                        better = d2 < bmins[j]
                        bmins[j] = jnp.where(better, d2, bmins[j])
                        bidxs[j] = jnp.where(better, idxvec, bidxs[j])
                return tuple(bmins), tuple(bidxs)

            inf16 = jnp.full((L,), jnp.inf, dtype=jnp.float32)
            z16 = jnp.zeros((L,), dtype=jnp.int32)
            bmins, bidxs = lax.fori_loop(
                0, m_sc // (L * SC_UNROLL), body,
                (tuple(inf16 for _ in range(SC_QT)),
                 tuple(z16 for _ in range(SC_QT))))

            for j in range(SC_QT):
                rminv[qt * SC_QT + j, :] = bmins[j]
                ridxv[qt * SC_QT + j, :] = bidxs[j]

        pltpu.sync_copy(rminv, omin_h.at[pl.ds(qbase, qpw)])
        pltpu.sync_copy(ridxv, oidx_h.at[pl.ds(qbase, qpw)])

    return sc_nn


def _tc_dist_kernel(sT_ref, map_ref, min_ref, arg_ref, *, n_pts, m_tc):
    N = n_pts
    C = CHUNK
    sT = sT_ref[0]
    sx = sT[0:1, :]
    sy = sT[1:2, :]
    sz = sT[2:3, :]

    def chunk_body(ci, carry):
        run_min, run_arg = carry
        m = map_ref[0, pl.ds(ci * C, C), :]
        dx = m[:, 0:1] - sx
        dy = m[:, 1:2] - sy
        dz = m[:, 2:3] - sz
        d2 = dx * dx + dy * dy + dz * dz
        tmin = jnp.min(d2, axis=0, keepdims=True)
        iota = lax.broadcasted_iota(jnp.int32, (C, N), 0) + ci * C
        targ = jnp.min(jnp.where(d2 == tmin, iota, m_tc),
                       axis=0, keepdims=True)
        better = tmin < run_min
        return (jnp.where(better, tmin, run_min),
                jnp.where(better, targ, run_arg))

    run_min0 = jnp.full((1, N), jnp.inf, dtype=jnp.float32)
    run_arg0 = jnp.zeros((1, N), dtype=jnp.int32)
    rmin, rarg = lax.fori_loop(0, m_tc // C, chunk_body, (run_min0, run_arg0))
    min_ref[0] = rmin
    arg_ref[0] = rarg


def _tc_stats_kernel(sT_ref, inten_ref, mapT_ref, min1_ref, arg1_ref,
                     scmin_ref, scidx_ref, T_ref, params_ref, out_ref,
                     *, n_pts, n_map, m_tc):
    N = n_pts
    M = n_map
    C = CHUNK

    sT = sT_ref[0]
    inten = inten_ref[0]
    T = T_ref[0]
    prm = params_ref[...]

    a = jnp.maximum(prm[0, 0], 0.0)
    b = jnp.maximum(prm[0, 1], 0.0)
    thresh = a * jnp.mean(inten) + b
    w_bfar = jax.nn.sigmoid((inten - thresh) * BFAR_TEMP)

    sx = sT[0:1, :]
    sy = sT[1:2, :]
    sz = sT[2:3, :]

    pm = scmin_ref[0]
    pi = scidx_ref[0]
    gmin2 = jnp.min(pm, axis=1, keepdims=True)
    cand = jnp.where(pm == gmin2, pi, M)
    gidx2 = jnp.min(cand, axis=1, keepdims=True)
    m2 = jnp.transpose(gmin2)
    a2 = jnp.transpose(gidx2) + m_tc

    m1 = min1_ref[0]
    a1 = arg1_ref[0]
    take2 = (m2 < m1) | ((m2 == m1) & (a2 < a1))
    idx = jnp.where(take2, a2, a1)

    def gather_body(ci, nn, idx=idx):
        iota = lax.broadcasted_iota(jnp.int32, (C, N), 0) + ci * C
        onehot = (iota == idx).astype(jnp.float32)
        mT = mapT_ref[0, :, pl.ds(ci * C, C)]
        return nn + lax.dot_general(
            mT, onehot, (((1,), (0,)), ((), ())),
            preferred_element_type=jnp.float32)

    nn = lax.fori_loop(0, M // C, gather_body,
                       jnp.zeros((3, N), dtype=jnp.float32))

    nx = nn[0:1, :]
    ny = nn[1:2, :]
    nz = nn[2:3, :]
    rx = nx - sx
    ry = ny - sy
    rz = nz - sz
    d2r = rx * rx + ry * ry + rz * rz
    dist = jnp.sqrt(d2r + 1e-12)
    w_h = jnp.where(dist < HUBER_DELTA, 1.0, HUBER_DELTA / dist)
    w = w_bfar * w_h * (dist < TRIM_DIST).astype(jnp.float32)
    wsum = jnp.sum(w) + 1e-8

    mu_sx = jnp.sum(w * sx) / wsum
    mu_sy = jnp.sum(w * sy) / wsum
    mu_mx = jnp.sum(w * nx) / wsum
    mu_my = jnp.sum(w * ny) / wsum
    sc0 = sx - mu_sx
    sc1 = sy - mu_sy
    mc0 = nx - mu_mx
    mc1 = ny - mu_my
    cross = jnp.sum(w * (sc0 * mc1 - sc1 * mc0))
    dot = jnp.sum(w * (sc0 * mc0 + sc1 * mc1))
    h = jnp.sqrt(cross * cross + dot * dot)
    safe = h > 0.0
    c = jnp.where(safe, dot / jnp.where(safe, h, 1.0), 1.0)
    sn = jnp.where(safe, cross / jnp.where(safe, h, 1.0), 0.0)
    t2x = mu_mx - (c * mu_sx - sn * mu_sy)
    t2y = mu_my - (sn * mu_sx + c * mu_sy)

    row0 = c * T[0:1, :] - sn * T[1:2, :] + t2x * T[3:4, :]
    row1 = sn * T[0:1, :] + c * T[1:2, :] + t2y * T[3:4, :]
    out_ref[0] = jnp.concatenate([row0, row1, T[2:3, :], T[3:4, :]], axis=0)


def kernel(scan_pc, scan_intensity, map_pc, T_init, params):
    B, N, _ = scan_pc.shape
    M = map_pc.shape[1]
    m_tc = M - M_SC
    NQ = B * N

    mapT = jnp.transpose(map_pc, (0, 2, 1))
    map_tc = map_pc[:, :m_tc, :]
    sc_mx = mapT[:, 0, m_tc:].reshape(-1)
    sc_my = mapT[:, 1, m_tc:].reshape(-1)
    sc_mz = mapT[:, 2, m_tc:].reshape(-1)
    inten3 = scan_intensity[:, None, :]
    prm2 = params.reshape(1, 2)

    sc_nn = _make_sc_nn(NQ, M_SC, N)

    dist_call = pl.pallas_call(
        functools.partial(_tc_dist_kernel, n_pts=N, m_tc=m_tc),
        grid=(B,),
        in_specs=[
            pl.BlockSpec((1, 3, N), lambda i: (i, 0, 0)),
            pl.BlockSpec((1, m_tc, 3), lambda i: (i, 0, 0)),
        ],
        out_specs=[
            pl.BlockSpec((1, 1, N), lambda i: (i, 0, 0)),
            pl.BlockSpec((1, 1, N), lambda i: (i, 0, 0)),
        ],
        out_shape=[
            jax.ShapeDtypeStruct((B, 1, N), jnp.float32),
            jax.ShapeDtypeStruct((B, 1, N), jnp.int32),
        ],
    )

    stats_call = pl.pallas_call(
        functools.partial(_tc_stats_kernel, n_pts=N, n_map=M, m_tc=m_tc),
        grid=(B,),
        in_specs=[
            pl.BlockSpec((1, 3, N), lambda i: (i, 0, 0)),
            pl.BlockSpec((1, 1, N), lambda i: (i, 0, 0)),
            pl.BlockSpec((1, 3, M), lambda i: (i, 0, 0)),
            pl.BlockSpec((1, 1, N), lambda i: (i, 0, 0)),
            pl.BlockSpec((1, 1, N), lambda i: (i, 0, 0)),
            pl.BlockSpec((1, N, L), lambda i: (i, 0, 0)),
            pl.BlockSpec((1, N, L), lambda i: (i, 0, 0)),
            pl.BlockSpec((1, 4, 4), lambda i: (i, 0, 0)),
            pl.BlockSpec((1, 2), lambda i: (0, 0)),
        ],
        out_specs=pl.BlockSpec((1, 4, 4), lambda i: (i, 0, 0)),
        out_shape=jax.ShapeDtypeStruct((B, 4, 4), jnp.float32),
    )

    T = T_init
    for _ in range(ICP_ITERS):
        R = T[:, :3, :3]
        t = T[:, :3, 3]
        s = jnp.einsum('bnk,bjk->bnj', scan_pc, R) + t[:, None, :]
        sT = jnp.transpose(s, (0, 2, 1))
        sq = s.reshape(NQ, 3, 1)
        qb = jnp.broadcast_to(sq, (NQ, 3, L))
        qxb = qb[:, 0, :]
        qyb = qb[:, 1, :]
        qzb = qb[:, 2, :]

        pmin, pidx = sc_nn(qxb, qyb, qzb, sc_mx, sc_my, sc_mz)
        min1, arg1 = dist_call(sT, map_tc)
        T = stats_call(sT, inten3, mapT, min1, arg1,
                       pmin.reshape(B, N, L), pidx.reshape(B, N, L),
                       T, prm2)
    return T
